# baseline (speedup 1.0000x reference)
import functools

import jax
import jax.numpy as jnp
from jax import lax
from jax.experimental import pallas as pl
from jax.experimental.pallas import tpu as pltpu
from jax.experimental.pallas import tpu_sc as plsc

VOCAB = 30522
D = 768
B = 128
N = 50
L = 32
EPS = 1e-12

LANES = 16
NV = D // LANES
NC = 2
NS = 16
NW = NC * NS
T = B * N * L
TPW = T // NW
CB = L
NCHUNK = TPW // CB
NB = 4
NGROUP = NCHUNK // NB

_MAGIC = 0x5F3759DF


def _lane_sum(x):
    i = lax.iota(jnp.int32, LANES)
    for k in (8, 4, 2, 1):
        x = x + x.at[i ^ k].get(mode="promise_in_bounds")
    return x


def _ln_chunk(rows_ref, pos_ref, gb_refs, a_ref, o_ref):
    inv_d = jnp.float32(1.0 / D)
    c15 = jnp.float32(1.5)

    @plsc.parallel_loop(0, CB, 1, unroll=4)
    def stats_body(t):
        zero = jnp.zeros((LANES,), jnp.float32)
        s = [zero, zero, zero, zero]
        q = [zero, zero, zero, zero]
        for c in range(NV):
            sl = pl.ds(c * LANES, LANES)
            x = rows_ref[t, sl] + pos_ref[t, sl]
            rows_ref[t, sl] = x
            s[c % 4] = s[c % 4] + x
            q[c % 4] = q[c % 4] + x * x
        mean_v = _lane_sum((s[0] + s[1]) + (s[2] + s[3])) * inv_d
        meansq_v = _lane_sum((q[0] + q[1]) + (q[2] + q[3])) * inv_d
        v = meansq_v - mean_v * mean_v + jnp.float32(EPS)
        y = plsc.bitcast(
            jnp.int32(_MAGIC)
            - lax.shift_right_arithmetic(plsc.bitcast(v, jnp.int32), 1),
            jnp.float32)
        hv = jnp.float32(-0.5) * v
        y = y * (c15 + hv * y * y)
        y = y * (c15 + hv * y * y)
        a_ref[pl.ds(t * LANES, LANES)] = y
        o_ref[pl.ds(t * LANES, LANES)] = jnp.float32(0.0) - mean_v * y

    @plsc.parallel_loop(0, CB, 1, unroll=2)
    def norm_body(t):
        a = a_ref[pl.ds(t * LANES, LANES)]
        o = o_ref[pl.ds(t * LANES, LANES)]
        for c in range(NV):
            sl = pl.ds(c * LANES, LANES)
            rows_ref[t, sl] = rows_ref[t, sl] * a + o

    if gb_refs is None:
        return
    g_ref, b_ref = gb_refs

    @plsc.parallel_loop(0, NV, 1)
    def col_body(c):
        sl = pl.ds(c * LANES, LANES)
        g = g_ref[sl]
        b = b_ref[sl]
        for t in range(CB):
            rows_ref[t, sl] = rows_ref[t, sl] * g + b


def _make_sc_kernel(apply_gb):
    mesh = plsc.VectorSubcoreMesh(core_axis_name="c", subcore_axis_name="s")
    n_gb = 2 if apply_gb else 0

    @functools.partial(
        pl.kernel,
        out_type=jax.ShapeDtypeStruct((T, D), jnp.float32),
        mesh=mesh,
        compiler_params=pltpu.CompilerParams(needs_layout_passes=False),
        scratch_types=(
            [pltpu.VMEM((CB,), jnp.int32) for _ in range(NB)]
            + [pltpu.VMEM((CB, D), jnp.float32) for _ in range(NB)]
            + [pltpu.VMEM((L, D), jnp.float32)]
            + [pltpu.VMEM((CB * LANES,), jnp.float32),
               pltpu.VMEM((CB * LANES,), jnp.float32)]
            + [pltpu.VMEM((D,), jnp.float32) for _ in range(n_gb)]
            + [pltpu.SemaphoreType.DMA for _ in range(2 * NB)]
        ),
    )
    def sc_kernel(idx_hbm, table_hbm, pos_hbm, *rest):
        gb_hbm = rest[:n_gb]
        out_hbm = rest[n_gb]
        refs = rest[n_gb + 1:]
        idxs = list(refs[0:NB])
        rows = list(refs[NB:2 * NB])
        pos_v = refs[2 * NB]
        a_ref, o_ref = refs[2 * NB + 1:2 * NB + 3]
        gb_refs = tuple(refs[2 * NB + 3:2 * NB + 3 + n_gb]) or None
        sems = refs[2 * NB + 3 + n_gb:]
        gsems = list(sems[0:NB])
        ssems = list(sems[NB:2 * NB])

        wid = lax.axis_index("s") * NC + lax.axis_index("c")
        base = wid * TPW
        pltpu.sync_copy(pos_hbm, pos_v)
        if apply_gb:
            pltpu.sync_copy(gb_hbm[0], gb_refs[0])
            pltpu.sync_copy(gb_hbm[1], gb_refs[1])

        def fetch(chunk, j):
            off = base + chunk * CB
            pltpu.sync_copy(idx_hbm.at[pl.ds(off, CB)], idxs[j])
            pltpu.async_copy(table_hbm.at[idxs[j]], rows[j], gsems[j])

        def wait_fetch(j):
            pltpu.make_async_copy(table_hbm.at[idxs[j]], rows[j],
                                  gsems[j]).wait()

        def store(chunk, j):
            off = base + chunk * CB
            pltpu.make_async_copy(rows[j], out_hbm.at[pl.ds(off, CB)],
                                  ssems[j]).start()

        def wait_store(j):
            pltpu.make_async_copy(rows[j], out_hbm.at[pl.ds(0, CB)],
                                  ssems[j]).wait()

        def body(k, j):
            jn = (j + 1) % NB
            @pl.when(k >= NB - 1)
            def _():
                wait_store(jn)

            fetch(jnp.minimum(k + 1, NCHUNK - 1), jn)
            wait_fetch(j)
            _ln_chunk(rows[j], pos_v, gb_refs, a_ref, o_ref)
            store(k, j)

        fetch(0, 0)

        def group_body(g, carry):
            k = NB * g
            for r in range(NB):
                body(k + r, r)
            return carry

        lax.fori_loop(0, NGROUP, group_body, 0)
        wait_fetch(0)
        for j in range(1, NB):
            wait_store(j)

    return sc_kernel


_SC_FAST = _make_sc_kernel(apply_gb=False)
_SC_FULL = _make_sc_kernel(apply_gb=True)


def kernel(news_batch, table, pos_emb, gamma, beta):
    idx = news_batch.reshape(T).astype(jnp.int32)
    identity_gb = jnp.logical_and(jnp.all(gamma == jnp.float32(1.0)),
                                  jnp.all(beta == jnp.float32(0.0)))
    out = lax.cond(
        identity_gb,
        lambda: _SC_FAST(idx, table, pos_emb),
        lambda: _SC_FULL(idx, table, pos_emb, gamma, beta),
    )
    return out.reshape(B, N, L, D)


# speedup vs baseline: 1.0609x; 1.0609x over previous
import functools

import jax
import jax.numpy as jnp
from jax import lax
from jax.experimental import pallas as pl
from jax.experimental.pallas import tpu as pltpu
from jax.experimental.pallas import tpu_sc as plsc

VOCAB = 30522
D = 768
B = 128
N = 50
L = 32
EPS = 1e-12

LANES = 16
NV = D // LANES
NC = 2
NS = 16
NW = NC * NS
T = B * N * L
TPW = T // NW
CB = L
NCHUNK = TPW // CB
NB = 4
NGROUP = NCHUNK // NB

_MAGIC = 0x5F3759DF


def _lane_sum(x):
    i = lax.iota(jnp.int32, LANES)
    for k in (8, 4, 2, 1):
        x = x + x.at[i ^ k].get(mode="promise_in_bounds")
    return x


def _ln_chunk(rows_ref, pos_ref, gb_refs):
    inv_d = jnp.float32(1.0 / D)
    c15 = jnp.float32(1.5)

    @plsc.parallel_loop(0, CB, 1, unroll=2)
    def token_body(t):
        zero = jnp.zeros((LANES,), jnp.float32)
        s = [zero, zero, zero, zero]
        q = [zero, zero, zero, zero]
        for c in range(NV):
            sl = pl.ds(c * LANES, LANES)
            x = rows_ref[t, sl] + pos_ref[t, sl]
            rows_ref[t, sl] = x
            s[c % 4] = s[c % 4] + x
            q[c % 4] = q[c % 4] + x * x
        mean_v = _lane_sum((s[0] + s[1]) + (s[2] + s[3])) * inv_d
        meansq_v = _lane_sum((q[0] + q[1]) + (q[2] + q[3])) * inv_d
        v = meansq_v - mean_v * mean_v + jnp.float32(EPS)
        y = plsc.bitcast(
            jnp.int32(_MAGIC)
            - lax.shift_right_arithmetic(plsc.bitcast(v, jnp.int32), 1),
            jnp.float32)
        hv = jnp.float32(-0.5) * v
        y = y * (c15 + hv * y * y)
        y = y * (c15 + hv * y * y)
        o = jnp.float32(0.0) - mean_v * y
        for c in range(NV):
            sl = pl.ds(c * LANES, LANES)
            rows_ref[t, sl] = rows_ref[t, sl] * y + o

    if gb_refs is None:
        return
    g_ref, b_ref = gb_refs

    @plsc.parallel_loop(0, NV, 1)
    def col_body(c):
        sl = pl.ds(c * LANES, LANES)
        g = g_ref[sl]
        b = b_ref[sl]
        for t in range(CB):
            rows_ref[t, sl] = rows_ref[t, sl] * g + b


def _make_sc_kernel(apply_gb):
    mesh = plsc.VectorSubcoreMesh(core_axis_name="c", subcore_axis_name="s")
    n_gb = 2 if apply_gb else 0

    @functools.partial(
        pl.kernel,
        out_type=jax.ShapeDtypeStruct((T, D), jnp.float32),
        mesh=mesh,
        compiler_params=pltpu.CompilerParams(needs_layout_passes=False),
        scratch_types=(
            [pltpu.VMEM((CB,), jnp.int32) for _ in range(NB)]
            + [pltpu.VMEM((CB, D), jnp.float32) for _ in range(NB)]
            + [pltpu.VMEM((L, D), jnp.float32)]
            + [pltpu.VMEM((D,), jnp.float32) for _ in range(n_gb)]
            + [pltpu.SemaphoreType.DMA for _ in range(2 * NB)]
        ),
    )
    def sc_kernel(idx_hbm, table_hbm, pos_hbm, *rest):
        gb_hbm = rest[:n_gb]
        out_hbm = rest[n_gb]
        refs = rest[n_gb + 1:]
        idxs = list(refs[0:NB])
        rows = list(refs[NB:2 * NB])
        pos_v = refs[2 * NB]
        gb_refs = tuple(refs[2 * NB + 1:2 * NB + 1 + n_gb]) or None
        sems = refs[2 * NB + 1 + n_gb:]
        gsems = list(sems[0:NB])
        ssems = list(sems[NB:2 * NB])

        wid = lax.axis_index("s") * NC + lax.axis_index("c")
        base = wid * TPW
        pltpu.sync_copy(pos_hbm, pos_v)
        if apply_gb:
            pltpu.sync_copy(gb_hbm[0], gb_refs[0])
            pltpu.sync_copy(gb_hbm[1], gb_refs[1])

        def fetch(chunk, j):
            off = base + chunk * CB
            pltpu.sync_copy(idx_hbm.at[pl.ds(off, CB)], idxs[j])
            pltpu.async_copy(table_hbm.at[idxs[j]], rows[j], gsems[j])

        def wait_fetch(j):
            pltpu.make_async_copy(table_hbm.at[idxs[j]], rows[j],
                                  gsems[j]).wait()

        def store(chunk, j):
            off = base + chunk * CB
            pltpu.make_async_copy(rows[j], out_hbm.at[pl.ds(off, CB)],
                                  ssems[j]).start()

        def wait_store(j):
            pltpu.make_async_copy(rows[j], out_hbm.at[pl.ds(0, CB)],
                                  ssems[j]).wait()

        def body(k, j):
            jn = (j + 1) % NB
            @pl.when(k >= NB - 1)
            def _():
                wait_store(jn)

            fetch(jnp.minimum(k + 1, NCHUNK - 1), jn)
            wait_fetch(j)
            _ln_chunk(rows[j], pos_v, gb_refs)
            store(k, j)

        fetch(0, 0)

        def group_body(g, carry):
            k = NB * g
            for r in range(NB):
                body(k + r, r)
            return carry

        lax.fori_loop(0, NGROUP, group_body, 0)
        wait_fetch(0)
        for j in range(1, NB):
            wait_store(j)

    return sc_kernel


_SC_FAST = _make_sc_kernel(apply_gb=False)
_SC_FULL = _make_sc_kernel(apply_gb=True)


def kernel(news_batch, table, pos_emb, gamma, beta):
    idx = news_batch.reshape(T).astype(jnp.int32)
    identity_gb = jnp.logical_and(jnp.all(gamma == jnp.float32(1.0)),
                                  jnp.all(beta == jnp.float32(0.0)))
    out = lax.cond(
        identity_gb,
        lambda: _SC_FAST(idx, table, pos_emb),
        lambda: _SC_FULL(idx, table, pos_emb, gamma, beta),
    )
    return out.reshape(B, N, L, D)
